# baseline (speedup 1.0000x reference)
import functools

import jax
import jax.numpy as jnp
from jax import lax
from jax.experimental import pallas as pl
from jax.experimental.pallas import tpu as pltpu
from jax.experimental.pallas import tpu_sc as plsc

_N = 50000
_D = 64
_DH = 32
_E = 800000

_CHUNK = 128
_CPB = 8
_BLKE = _CPB * _CHUNK
_NBLK = 48
_EPT = _E // 16
_TAIL = _EPT - _NBLK * _BLKE
_TFULL = _TAIL // _CHUNK
_TREM = _TAIL - _TFULL * _CHUNK
_NPAD = 51200
_NROWS = _NPAD
_RPT = _NROWS // 16

_MBLK = 1600


def _build_segsum():
    mesh = plsc.VectorSubcoreMesh(core_axis_name="c", subcore_axis_name="s")

    @functools.partial(
        pl.kernel,
        out_type=jax.ShapeDtypeStruct((_NROWS, 2, _DH), jnp.float32),
        mesh=mesh,
        compiler_params=pltpu.CompilerParams(use_tc_tiling_on_sc=False),
        scratch_types=[
            pltpu.VMEM((2 * _BLKE,), jnp.int32),
            pltpu.VMEM((2 * _BLKE,), jnp.int32),
            pltpu.VMEM((_TAIL,), jnp.int32),
            pltpu.VMEM((_TAIL,), jnp.int32),
            pltpu.VMEM((4, _CHUNK, 1, _DH), jnp.float32),
            pltpu.VMEM_SHARED((_NROWS, 1, _DH), jnp.float32),
            pltpu.SemaphoreType.DMA,
            pltpu.SemaphoreType.DMA,
            pltpu.SemaphoreType.DMA,
        ],
    )
    def segsum(table, ei3, zeros_hbm, out,
               src_r, dst_r, tsrc, tdst, rows_r, agg, sem_i, sem_g, sem_s):
        c = lax.axis_index("c")
        s = lax.axis_index("s")
        base = s * _EPT

        pltpu.sync_copy(zeros_hbm, agg.at[pl.ds(s * _RPT, _RPT)])
        plsc.subcore_barrier()

        def run(row):
            def fetch_idx(b, slot):
                off = base + b * _BLKE
                pltpu.make_async_copy(ei3.at[row, pl.ds(off, _BLKE)],
                                      src_r.at[pl.ds(slot * _BLKE, _BLKE)],
                                      sem_i).start()
                pltpu.make_async_copy(ei3.at[2, pl.ds(off, _BLKE)],
                                      dst_r.at[pl.ds(slot * _BLKE, _BLKE)],
                                      sem_i).start()

            def wait_idx(slot):
                pltpu.make_async_copy(ei3.at[row, pl.ds(base, _BLKE)],
                                      src_r.at[pl.ds(slot * _BLKE, _BLKE)],
                                      sem_i).wait()
                pltpu.make_async_copy(ei3.at[2, pl.ds(base, _BLKE)],
                                      dst_r.at[pl.ds(slot * _BLKE, _BLKE)],
                                      sem_i).wait()

            def buf(j, n=_CHUNK):
                r = rows_r.at[j % 4]
                return r if n == _CHUNK else r.at[pl.ds(0, n)]

            def start_g(idx, j, n=_CHUNK):
                pltpu.make_async_copy(table.at[idx], buf(j, n), sem_g).start()

            def wait_g(j, n=_CHUNK):
                pltpu.make_async_copy(table.at[tsrc.at[pl.ds(0, n)]],
                                      buf(j, n), sem_g).wait()

            def start_s(idx, j, n=_CHUNK):
                pltpu.async_copy(buf(j, n), agg.at[idx], sem_s, add=True)

            def wait_s(j, n=_CHUNK):
                pltpu.make_async_copy(buf(j, n),
                                      agg.at[tdst.at[pl.ds(0, n)]],
                                      sem_s).wait()

            def chunk_idx(r, slot, j):
                return r.at[pl.ds(slot * _BLKE + j * _CHUNK, _CHUNK)]

            fetch_idx(0, 0)
            fetch_idx(1, 1)
            tail_off = base + _NBLK * _BLKE
            pltpu.sync_copy(ei3.at[row, pl.ds(tail_off, _TAIL)], tsrc)
            pltpu.sync_copy(ei3.at[2, pl.ds(tail_off, _TAIL)], tdst)

            def do_block(b, slot):
                wait_idx(slot)
                start_g(chunk_idx(src_r, slot, 0), 0)
                start_g(chunk_idx(src_r, slot, 1), 1)
                for j in range(_CPB):
                    wait_g(j)
                    start_s(chunk_idx(dst_r, slot, j), j)
                    if j >= 2:
                        wait_s(j - 2)
                    if j + 2 < _CPB:
                        start_g(chunk_idx(src_r, slot, j + 2), j + 2)
                wait_s(_CPB - 2)
                wait_s(_CPB - 1)

                @pl.when(b + 2 < _NBLK)
                def _():
                    fetch_idx(b + 2, slot)

            def body(k, carry):
                do_block(2 * k, 0)
                do_block(2 * k + 1, 1)
                return carry

            lax.fori_loop(0, _NBLK // 2, body, 0)

            sizes = [_CHUNK] * _TFULL + [_TREM]
            nt = len(sizes)

            def tidx(r, j, n):
                return r.at[pl.ds(j * _CHUNK, n)]

            start_g(tidx(tsrc, 0, sizes[0]), 0, sizes[0])
            start_g(tidx(tsrc, 1, sizes[1]), 1, sizes[1])
            for j in range(nt):
                wait_g(j, sizes[j])
                start_s(tidx(tdst, j, sizes[j]), j, sizes[j])
                if j >= 2:
                    wait_s(j - 2, sizes[j - 2])
                if j + 2 < nt:
                    start_g(tidx(tsrc, j + 2, sizes[j + 2]), j + 2,
                            sizes[j + 2])
            wait_s(nt - 2, sizes[nt - 2])
            wait_s(nt - 1, sizes[nt - 1])

        @pl.when(c == 0)
        def _():
            run(0)

        @pl.when(c == 1)
        def _():
            run(1)

        plsc.subcore_barrier()

        @pl.when(c == 0)
        def _():
            pltpu.sync_copy(agg.at[pl.ds(s * _RPT, _RPT)],
                            out.at[pl.ds(s * _RPT, _RPT), pl.ds(0, 1)])

        @pl.when(c == 1)
        def _():
            pltpu.sync_copy(agg.at[pl.ds(s * _RPT, _RPT)],
                            out.at[pl.ds(s * _RPT, _RPT), pl.ds(1, 1)])

    return segsum


_segsum = _build_segsum()


def _mlp_body(xv_r, av_r, w1_r, b1_r, w2_r, b2_r, out_r):
    h = xv_r[...] + av_r[...]
    t = jnp.tanh(jnp.dot(h, w1_r[...], preferred_element_type=jnp.float32)
                 + b1_r[...])
    out_r[...] = (jnp.dot(t, w2_r[...], preferred_element_type=jnp.float32)
                  + b2_r[...])


def _mlp(xv, av, w1d, b1d, w2d, b2d):
    row_spec = pl.BlockSpec((_MBLK, 128), lambda i: (i, 0))
    full_spec = pl.BlockSpec((128, 128), lambda i: (0, 0))
    bias_spec = pl.BlockSpec((1, 128), lambda i: (0, 0))
    return pl.pallas_call(
        _mlp_body,
        grid=(_NPAD // 2 // _MBLK,),
        in_specs=[row_spec, row_spec,
                  full_spec, bias_spec, full_spec, bias_spec],
        out_specs=row_spec,
        out_shape=jax.ShapeDtypeStruct((_NPAD // 2, 128), jnp.float32),
    )(xv, av, w1d, b1d, w2d, b2d)


def _blockdiag(w):
    z = jnp.zeros((_D, _D), jnp.float32)
    return jnp.block([[w.T, z], [z, w.T]])


def _prep_edges(edge_index):
    ei = edge_index.astype(jnp.int32)
    src2 = ei[0] * 2
    return jnp.stack([src2, src2 + 1, ei[1]])


def kernel(x, edge_index0, edge_index1, W1_0, b1_0, W2_0, b2_0,
           W1_1, b1_1, W2_1, b2_1):
    x = x.astype(jnp.float32)
    xp = jnp.pad(x, ((0, _NPAD - _N), (0, 0)))
    xv = xp.reshape(_NPAD // 2, 128)
    e0 = _prep_edges(edge_index0)
    e1 = _prep_edges(edge_index1)
    zeros = jnp.zeros((_RPT, 1, _DH), jnp.float32)

    a0 = _segsum(xv.reshape(2 * _NPAD, 1, _DH), e0, zeros)
    hv = _mlp(xv, a0.reshape(_NPAD // 2, 128),
              _blockdiag(W1_0), jnp.tile(b1_0, 2).reshape(1, 128),
              _blockdiag(W2_0), jnp.tile(b2_0, 2).reshape(1, 128))
    a1 = _segsum(hv.reshape(2 * _NPAD, 1, _DH), e1, zeros)
    ov = _mlp(hv, a1.reshape(_NPAD // 2, 128),
              _blockdiag(W1_1), jnp.tile(b1_1, 2).reshape(1, 128),
              _blockdiag(W2_1), jnp.tile(b2_1, 2).reshape(1, 128))
    return ov.reshape(_NPAD, _D)[:_N]


# speedup vs baseline: 2.5428x; 2.5428x over previous
import functools

import jax
import jax.numpy as jnp
from jax import lax
from jax.experimental import pallas as pl
from jax.experimental.pallas import tpu as pltpu
from jax.experimental.pallas import tpu_sc as plsc

_N = 50000
_D = 64
_DH = 32
_E = 800000

_CHUNK = 128
_CPB = 8
_BLKE = _CPB * _CHUNK
_NBLK = 48
_EPT = _E // 16
_TAIL = _EPT - _NBLK * _BLKE
_TFULL = _TAIL // _CHUNK
_TREM = _TAIL - _TFULL * _CHUNK
_NPAD = 51200
_NROWS = _NPAD
_RPT = _NROWS // 16

_MBLK = 1600


def _build_segsum():
    mesh = plsc.VectorSubcoreMesh(core_axis_name="c", subcore_axis_name="s")

    @functools.partial(
        pl.kernel,
        out_type=jax.ShapeDtypeStruct((_NPAD // 2, 128), jnp.float32),
        mesh=mesh,
        compiler_params=pltpu.CompilerParams(use_tc_tiling_on_sc=False),
        scratch_types=[
            pltpu.VMEM((2 * _BLKE,), jnp.int32),
            pltpu.VMEM((2 * _BLKE,), jnp.int32),
            pltpu.VMEM((_TAIL,), jnp.int32),
            pltpu.VMEM((_TAIL,), jnp.int32),
            pltpu.VMEM((4, _CHUNK, _DH), jnp.float32),
            pltpu.VMEM_SHARED((_NROWS, _DH), jnp.float32),
            pltpu.SemaphoreType.DMA,
            pltpu.SemaphoreType.DMA,
            pltpu.SemaphoreType.DMA,
        ],
    )
    def segsum(table, ei3, zeros_hbm, out,
               src_r, dst_r, tsrc, tdst, rows_r, agg, sem_i, sem_g, sem_s):
        c = lax.axis_index("c")
        s = lax.axis_index("s")
        base = s * _EPT

        pltpu.sync_copy(zeros_hbm, agg.at[pl.ds(s * _RPT, _RPT)])
        plsc.subcore_barrier()

        def run(row):
            def fetch_idx(b, slot):
                off = base + b * _BLKE
                pltpu.make_async_copy(ei3.at[row, pl.ds(off, _BLKE)],
                                      src_r.at[pl.ds(slot * _BLKE, _BLKE)],
                                      sem_i).start()
                pltpu.make_async_copy(ei3.at[2, pl.ds(off, _BLKE)],
                                      dst_r.at[pl.ds(slot * _BLKE, _BLKE)],
                                      sem_i).start()

            def wait_idx(slot):
                pltpu.make_async_copy(ei3.at[row, pl.ds(base, _BLKE)],
                                      src_r.at[pl.ds(slot * _BLKE, _BLKE)],
                                      sem_i).wait()
                pltpu.make_async_copy(ei3.at[2, pl.ds(base, _BLKE)],
                                      dst_r.at[pl.ds(slot * _BLKE, _BLKE)],
                                      sem_i).wait()

            def buf(j, n=_CHUNK):
                r = rows_r.at[j % 4]
                return r if n == _CHUNK else r.at[pl.ds(0, n)]

            def start_g(idx, j, n=_CHUNK):
                pltpu.make_async_copy(table.at[idx], buf(j, n), sem_g).start()

            def wait_g(j, n=_CHUNK):
                pltpu.make_async_copy(table.at[tsrc.at[pl.ds(0, n)]],
                                      buf(j, n), sem_g).wait()

            def start_s(idx, j, n=_CHUNK):
                pltpu.async_copy(buf(j, n), agg.at[idx], sem_s, add=True)

            def wait_s(j, n=_CHUNK):
                pltpu.make_async_copy(buf(j, n),
                                      agg.at[tdst.at[pl.ds(0, n)]],
                                      sem_s).wait()

            def chunk_idx(r, slot, j):
                return r.at[pl.ds(slot * _BLKE + j * _CHUNK, _CHUNK)]

            fetch_idx(0, 0)
            fetch_idx(1, 1)
            tail_off = base + _NBLK * _BLKE
            pltpu.sync_copy(ei3.at[row, pl.ds(tail_off, _TAIL)], tsrc)
            pltpu.sync_copy(ei3.at[2, pl.ds(tail_off, _TAIL)], tdst)

            def do_block(b, slot):
                wait_idx(slot)
                start_g(chunk_idx(src_r, slot, 0), 0)
                start_g(chunk_idx(src_r, slot, 1), 1)
                for j in range(_CPB):
                    wait_g(j)
                    start_s(chunk_idx(dst_r, slot, j), j)
                    if j >= 2:
                        wait_s(j - 2)
                    if j + 2 < _CPB:
                        start_g(chunk_idx(src_r, slot, j + 2), j + 2)
                wait_s(_CPB - 2)
                wait_s(_CPB - 1)

                @pl.when(b + 2 < _NBLK)
                def _():
                    fetch_idx(b + 2, slot)

            def body(k, carry):
                do_block(2 * k, 0)
                do_block(2 * k + 1, 1)
                return carry

            lax.fori_loop(0, _NBLK // 2, body, 0)

            sizes = [_CHUNK] * _TFULL + [_TREM]
            nt = len(sizes)

            def tidx(r, j, n):
                return r.at[pl.ds(j * _CHUNK, n)]

            start_g(tidx(tsrc, 0, sizes[0]), 0, sizes[0])
            start_g(tidx(tsrc, 1, sizes[1]), 1, sizes[1])
            for j in range(nt):
                wait_g(j, sizes[j])
                start_s(tidx(tdst, j, sizes[j]), j, sizes[j])
                if j >= 2:
                    wait_s(j - 2, sizes[j - 2])
                if j + 2 < nt:
                    start_g(tidx(tsrc, j + 2, sizes[j + 2]), j + 2,
                            sizes[j + 2])
            wait_s(nt - 2, sizes[nt - 2])
            wait_s(nt - 1, sizes[nt - 1])

        @pl.when(c == 0)
        def _():
            run(0)

        @pl.when(c == 1)
        def _():
            run(1)

        plsc.subcore_barrier()

        h = _RPT // 2

        def wb(arow, ocol):
            pltpu.sync_copy(agg.at[pl.ds(arow, h)],
                            out.at[pl.ds(s * h, h), pl.ds(ocol, _DH)])

        @pl.when(c == 0)
        def _():
            wb(s * h, 0)
            wb(_NROWS // 2 + s * h, 2 * _DH)

        @pl.when(c == 1)
        def _():
            wb(s * h, _DH)
            wb(_NROWS // 2 + s * h, 3 * _DH)

    return segsum


_segsum = _build_segsum()


def _mlp_body(xv_r, av_r, w1_r, b1_r, w2_r, b2_r, out_r):
    h = xv_r[...] + av_r[...]
    t = jnp.tanh(jnp.dot(h, w1_r[...], preferred_element_type=jnp.float32)
                 + b1_r[...])
    out_r[...] = (jnp.dot(t, w2_r[...], preferred_element_type=jnp.float32)
                  + b2_r[...])


def _mlp(xv, av, w1d, b1d, w2d, b2d):
    row_spec = pl.BlockSpec((_MBLK, 128), lambda i: (i, 0))
    full_spec = pl.BlockSpec((128, 128), lambda i: (0, 0))
    bias_spec = pl.BlockSpec((1, 128), lambda i: (0, 0))
    return pl.pallas_call(
        _mlp_body,
        grid=(_NPAD // 2 // _MBLK,),
        in_specs=[row_spec, row_spec,
                  full_spec, bias_spec, full_spec, bias_spec],
        out_specs=row_spec,
        out_shape=jax.ShapeDtypeStruct((_NPAD // 2, 128), jnp.float32),
    )(xv, av, w1d, b1d, w2d, b2d)


def _blockdiag(w):
    z = jnp.zeros((_D, _D), jnp.float32)
    return jnp.block([[w.T, z], [z, w.T]])


def _prep_edges(edge_index):
    ei = edge_index.astype(jnp.int32)
    src2 = ei[0] * 2
    dst = ei[1]
    dstp = (dst % 2) * (_NROWS // 2) + dst // 2
    return jnp.stack([src2, src2 + 1, dstp])


def kernel(x, edge_index0, edge_index1, W1_0, b1_0, W2_0, b2_0,
           W1_1, b1_1, W2_1, b2_1):
    x = x.astype(jnp.float32)
    xp = jnp.pad(x, ((0, _NPAD - _N), (0, 0)))
    xv = xp.reshape(_NPAD // 2, 128)
    e0 = _prep_edges(edge_index0)
    e1 = _prep_edges(edge_index1)
    zeros = jnp.zeros((_RPT, _DH), jnp.float32)

    a0 = _segsum(xv.reshape(2 * _NPAD, _DH), e0, zeros)
    hv = _mlp(xv, a0,
              _blockdiag(W1_0), jnp.tile(b1_0, 2).reshape(1, 128),
              _blockdiag(W2_0), jnp.tile(b2_0, 2).reshape(1, 128))
    a1 = _segsum(hv.reshape(2 * _NPAD, _DH), e1, zeros)
    ov = _mlp(hv, a1,
              _blockdiag(W1_1), jnp.tile(b1_1, 2).reshape(1, 128),
              _blockdiag(W2_1), jnp.tile(b2_1, 2).reshape(1, 128))
    return ov.reshape(_NPAD, _D)[:_N]
